# Initial kernel scaffold; baseline (speedup 1.0000x reference)
import jax
import jax.numpy as jnp
from jax.experimental import pallas as pl


def kernel(hidden_states, attention_mask, self_attention_scores, key_layer, tome_size, ln1_g, ln1_b, W1, b1, ln2_g, ln2_b, W2, b2):
    raise NotImplementedError("write your pallas kernel here")


# speedup vs baseline: 1.3026x; 1.3026x over previous
import functools
import math

import jax
import jax.numpy as jnp
from jax import lax
from jax.experimental import pallas as pl
from jax.experimental.pallas import tpu as pltpu


def _reduce_body(x_ref, o_ref):
    j = pl.program_id(1)

    @pl.when(j == 0)
    def _init():
        o_ref[...] = jnp.zeros_like(o_ref)

    o_ref[...] += jnp.sum(x_ref[...], axis=1, keepdims=True)


def _finish_body(bd_ref, sums_ref, hid_ref, ln1g_ref, ln1b_ref, w1_ref, b1_ref,
                 ln2g_ref, ln2b_ref, wd_ref, mask_ref, *, n_rows, eps):
    bd = bd_ref[0]
    L = sums_ref.shape[2]
    idx = lax.broadcasted_iota(jnp.int32, (1, L), 1)

    imp = sums_ref[0] * (1.0 / n_rows)
    imp_body = jnp.where(idx == 0, jnp.inf, imp)
    mn = jnp.min(imp_body)
    mx = jnp.max(jnp.where(idx == 0, -jnp.inf, imp))
    impn = (imp - mn) / mx
    impn = jnp.where(idx == 0, 1.0, impn)

    x = hid_ref[0]
    m = jnp.mean(x, axis=-1, keepdims=True)
    v = jnp.mean((x - m) ** 2, axis=-1, keepdims=True)
    x = (x - m) * lax.rsqrt(v + eps) * ln1g_ref[0] + ln1b_ref[0]

    h = jnp.dot(x, w1_ref[...], preferred_element_type=jnp.float32) + b1_ref[0]
    m = jnp.mean(h, axis=-1, keepdims=True)
    v = jnp.mean((h - m) ** 2, axis=-1, keepdims=True)
    h = (h - m) * lax.rsqrt(v + eps) * ln2g_ref[0] + ln2b_ref[0]
    h = 0.5 * h * (1.0 + lax.erf(h * (1.0 / math.sqrt(2.0))))

    diff = jnp.sum(h * wd_ref[0], axis=-1)[None, :] + bd
    diff = jnp.where(idx == 0, diff + 100.0, diff)
    learned = jax.nn.sigmoid(diff)

    final = (impn + learned) * 0.5
    mask_ref[0] = (final >= 0.5).astype(mask_ref.dtype)


def kernel(hidden_states, attention_mask, self_attention_scores, key_layer,
           tome_size, ln1_g, ln1_b, W1, b1, ln2_g, ln2_b, W2, b2):
    B, L, D = hidden_states.shape
    H = self_attention_scores.shape[1]
    n_rows = H * L

    scores = self_attention_scores.reshape(B, n_rows, L)
    R = 512
    nsteps = n_rows // R

    sums = pl.pallas_call(
        _reduce_body,
        grid=(B, nsteps),
        in_specs=[pl.BlockSpec((1, R, L), lambda b, j: (b, j, 0))],
        out_specs=pl.BlockSpec((1, 1, L), lambda b, j: (b, 0, 0)),
        out_shape=jax.ShapeDtypeStruct((B, 1, L), jnp.float32),
        compiler_params=pltpu.CompilerParams(
            dimension_semantics=("parallel", "arbitrary")),
    )(scores)

    w_diff = W2[:, 0] - W2[:, 1]
    b_diff = (b2[0] - b2[1]).astype(jnp.float32)

    mask = pl.pallas_call(
        functools.partial(_finish_body, n_rows=float(n_rows), eps=1e-5),
        grid=(B,),
        in_specs=[
            pl.BlockSpec(memory_space=pltpu.SMEM),
            pl.BlockSpec((1, 1, L), lambda b: (b, 0, 0)),
            pl.BlockSpec((1, L, D), lambda b: (b, 0, 0)),
            pl.BlockSpec((1, D), lambda b: (0, 0)),
            pl.BlockSpec((1, D), lambda b: (0, 0)),
            pl.BlockSpec((D, D), lambda b: (0, 0)),
            pl.BlockSpec((1, D), lambda b: (0, 0)),
            pl.BlockSpec((1, D), lambda b: (0, 0)),
            pl.BlockSpec((1, D), lambda b: (0, 0)),
            pl.BlockSpec((1, D), lambda b: (0, 0)),
        ],
        out_specs=pl.BlockSpec((1, 1, L), lambda b: (b, 0, 0)),
        out_shape=jax.ShapeDtypeStruct((B, 1, L), jnp.float32),
    )(b_diff[None], sums, hidden_states, ln1_g[None], ln1_b[None], W1, b1[None],
      ln2_g[None], ln2_b[None], w_diff[None])
    mask = mask.reshape(B, L)

    tome_size_new = jnp.ones((B, L, 1), dtype=attention_mask.dtype)
    return (hidden_states, attention_mask, tome_size_new, mask)
